# baseline (speedup 1.0000x reference)
import functools

import jax
import jax.numpy as jnp
from jax import lax
from jax.experimental import pallas as pl
from jax.experimental.pallas import tpu as pltpu
from jax.experimental.pallas import tpu_sc as plsc


def _build(batch: int, seq: int, hid: int):
    info = plsc.get_sparse_core_info()
    nc, ns, nl = info.num_cores, info.num_subcores, info.num_lanes
    nw = nc * ns
    bw = batch // nw
    assert batch == nw * bw and bw == 128
    nth = hid // 8
    assert hid == nth * 8 and nl == 16
    SC = 4
    assert seq % SC == 0

    mesh = plsc.VectorSubcoreMesh(core_axis_name="c", subcore_axis_name="s")

    @functools.partial(
        pl.kernel,
        mesh=mesh,
        compiler_params=pltpu.CompilerParams(
            use_tc_tiling_on_sc=False, needs_layout_passes=False
        ),
        out_type=jax.ShapeDtypeStruct((seq, nth, nw, 8, 128), jnp.float32),
        scratch_types=[
            pltpu.VMEM((bw, seq), jnp.int32),
            pltpu.VMEM((seq, hid), jnp.float32),
            pltpu.VMEM((3, SC * bw), jnp.int32),
            pltpu.VMEM((3, SC * bw, hid), jnp.float32),
            pltpu.VMEM((2, SC, nth, 8, 129), jnp.float32),
            pltpu.SemaphoreType.DMA,
            pltpu.SemaphoreType.DMA,
        ],
    )
    def emb_lookup(x_hbm, emb_hbm, pos_hbm, out_hbm,
                   xblk, pos_v, idx_v, rows_v, tile_v, gsem, osem):
        wid = lax.axis_index("s") * nc + lax.axis_index("c")
        pltpu.sync_copy(x_hbm.at[pl.ds(wid * bw, bw)], xblk)
        pltpu.sync_copy(pos_hbm, pos_v)

        iota = lax.iota(jnp.int32, 16)
        nstages = seq // SC
        zero16 = iota * 0
        hvecs = [iota + h0 for h0 in range(0, hid, 16)]

        def start_stage(g, buf):
            for k in range(SC):
                s = g * SC + k
                col = jnp.full((16,), s, jnp.int32)
                for i in range(bw // 16):
                    rows = iota + (i * 16)
                    idx_v[buf, pl.ds(k * bw + i * 16, 16)] = plsc.load_gather(
                        xblk, [rows, col]
                    )
            pltpu.async_copy(
                emb_hbm.at[idx_v.at[buf]], rows_v.at[buf], gsem
            )

        def wait_gathers(buf):
            pltpu.make_async_copy(
                emb_hbm.at[idx_v.at[buf]], rows_v.at[buf], gsem
            ).wait()

        start_stage(0, 0)
        start_stage(1, 1)

        def body(g, carry):
            p = lax.rem(g, 3)
            pt = lax.rem(g, 2)

            @pl.when(g + 2 < nstages)
            def _():
                start_stage(g + 2, lax.rem(g + 2, 3))

            wait_gathers(p)

            @pl.when(g >= 2)
            def _():
                pltpu.make_async_copy(
                    tile_v.at[0, :, :, :, pl.ds(0, 128)],
                    out_hbm.at[pl.ds(0, SC), :, wid],
                    osem,
                ).wait()

            for k in range(SC):
                s = g * SC + k
                pvs = [pos_v[s, pl.ds(h0, 16)] for h0 in range(0, hid, 16)]

                def tloop(lt, carry2):
                    lane0 = jnp.full((16,), lt * 16, jnp.int32)
                    for li in range(16):
                        l = lt * 16 + li
                        lane = lane0 + li
                        for j in range(hid // 16):
                            val = (
                                rows_v[p, k * bw + l, pl.ds(j * 16, 16)]
                                + pvs[j]
                            )
                            plsc.store_scatter(
                                tile_v.at[pt, k],
                                [zero16, hvecs[j], lane],
                                val,
                            )
                    return carry2

                lax.fori_loop(0, bw // 16, tloop, 0)

            pltpu.async_copy(
                tile_v.at[pt, :, :, :, pl.ds(0, 128)],
                out_hbm.at[pl.ds(g * SC, SC), :, wid],
                osem,
            )
            return carry

        lax.fori_loop(0, nstages, body, 0)
        for _ in range(2):
            pltpu.make_async_copy(
                tile_v.at[0, :, :, :, pl.ds(0, 128)],
                out_hbm.at[pl.ds(0, SC), :, wid],
                osem,
            ).wait()

    return emb_lookup, nth, nw


def kernel(x, emb, pos_emb):
    b, s = x.shape
    hid = emb.shape[1]
    fn, nth, nw = _build(b, s, hid)
    out6 = fn(x.astype(jnp.int32), emb, pos_emb)
    return out6.transpose(2, 4, 0, 1, 3).reshape(b, s, hid)


# speedup vs baseline: 1.0122x; 1.0122x over previous
import functools

import jax
import jax.numpy as jnp
from jax import lax
from jax.experimental import pallas as pl
from jax.experimental.pallas import tpu as pltpu
from jax.experimental.pallas import tpu_sc as plsc


def _build(batch: int, seq: int, hid: int):
    info = plsc.get_sparse_core_info()
    nc, ns, nl = info.num_cores, info.num_subcores, info.num_lanes
    nw = nc * ns
    bw = batch // nw
    assert batch == nw * bw and bw == 128
    nth = hid // 8
    assert hid == nth * 8 and nl == 16
    SC = 2
    assert seq % SC == 0

    mesh = plsc.VectorSubcoreMesh(core_axis_name="c", subcore_axis_name="s")

    @functools.partial(
        pl.kernel,
        mesh=mesh,
        compiler_params=pltpu.CompilerParams(
            use_tc_tiling_on_sc=False, needs_layout_passes=False
        ),
        out_type=jax.ShapeDtypeStruct((seq, nth, nw, 8, 128), jnp.float32),
        scratch_types=[
            pltpu.VMEM((bw, seq), jnp.int32),
            pltpu.VMEM((seq, hid), jnp.float32),
            pltpu.VMEM((3, SC * bw), jnp.int32),
            pltpu.VMEM((3, SC * bw, hid), jnp.float32),
            pltpu.VMEM((2, SC, nth, 8, 129), jnp.float32),
            pltpu.SemaphoreType.DMA,
            pltpu.SemaphoreType.DMA,
        ],
    )
    def emb_lookup(x_hbm, emb_hbm, pos_hbm, out_hbm,
                   xblk, pos_v, idx_v, rows_v, tile_v, gsem, osem):
        wid = lax.axis_index("s") * nc + lax.axis_index("c")
        pltpu.sync_copy(x_hbm.at[pl.ds(wid * bw, bw)], xblk)
        pltpu.sync_copy(pos_hbm, pos_v)

        iota = lax.iota(jnp.int32, 16)
        nstages = seq // SC
        zero16 = iota * 0
        hvecs = [iota + h0 for h0 in range(0, hid, 16)]

        def start_stage(g, buf):
            for k in range(SC):
                s = g * SC + k
                col = jnp.full((16,), s, jnp.int32)
                for i in range(bw // 16):
                    rows = iota + (i * 16)
                    idx_v[buf, pl.ds(k * bw + i * 16, 16)] = plsc.load_gather(
                        xblk, [rows, col]
                    )
            pltpu.async_copy(
                emb_hbm.at[idx_v.at[buf]], rows_v.at[buf], gsem
            )

        def wait_gathers(buf):
            pltpu.make_async_copy(
                emb_hbm.at[idx_v.at[buf]], rows_v.at[buf], gsem
            ).wait()

        start_stage(0, 0)
        start_stage(1, 1)

        def body(g, carry):
            p = lax.rem(g, 3)
            pt = lax.rem(g, 2)

            @pl.when(g + 2 < nstages)
            def _():
                start_stage(g + 2, lax.rem(g + 2, 3))

            wait_gathers(p)

            @pl.when(g >= 2)
            def _():
                pltpu.make_async_copy(
                    tile_v.at[0, :, :, :, pl.ds(0, 128)],
                    out_hbm.at[pl.ds(0, SC), :, wid],
                    osem,
                ).wait()

            for k in range(SC):
                s = g * SC + k
                pvs = [pos_v[s, pl.ds(h0, 16)] for h0 in range(0, hid, 16)]

                def tloop(lt, carry2):
                    lane0 = jnp.full((16,), lt * 16, jnp.int32)
                    for li in range(16):
                        l = lt * 16 + li
                        lane = lane0 + li
                        for j in range(hid // 16):
                            val = (
                                rows_v[p, k * bw + l, pl.ds(j * 16, 16)]
                                + pvs[j]
                            )
                            plsc.store_scatter(
                                tile_v.at[pt, k],
                                [zero16, hvecs[j], lane],
                                val,
                            )
                    return carry2

                lax.fori_loop(0, bw // 16, tloop, 0)

            pltpu.async_copy(
                tile_v.at[pt, :, :, :, pl.ds(0, 128)],
                out_hbm.at[pl.ds(g * SC, SC), :, wid],
                osem,
            )
            return carry

        lax.fori_loop(0, nstages, body, 0)
        for _ in range(2):
            pltpu.make_async_copy(
                tile_v.at[0, :, :, :, pl.ds(0, 128)],
                out_hbm.at[pl.ds(0, SC), :, wid],
                osem,
            ).wait()

    return emb_lookup, nth, nw


def kernel(x, emb, pos_emb):
    b, s = x.shape
    hid = emb.shape[1]
    fn, nth, nw = _build(b, s, hid)
    out6 = fn(x.astype(jnp.int32), emb, pos_emb)
    return out6.transpose(2, 4, 0, 1, 3).reshape(b, s, hid)
